# baseline (speedup 1.0000x reference)
import functools

import jax
import jax.numpy as jnp
import numpy as np
from jax import lax
from jax.experimental import pallas as pl
from jax.experimental.pallas import tpu as pltpu
from jax.experimental.pallas import tpu_sc as plsc

N_EMBD = 768
N_HID = 4 * N_EMBD
N_EXP = 64
_B, _T = 4, 8192
N_TOK = _B * _T

TILE = 1024


@functools.lru_cache(maxsize=1)
def _noise_const() -> np.ndarray:
    with jax.ensure_compile_time_eval():
        n = jax.random.normal(jax.random.key(42), (N_TOK, N_EXP),
                              dtype=jnp.float32)
    return np.ascontiguousarray(np.asarray(n).T)

NC, NS, LANES = 2, 16, 16
NW = NC * NS
TPW = N_TOK // NW
CH = 512
NCHUNK = TPW // CH
GPC = CH // LANES


def _score_body(x_ref, w1_ref, b1_ref, w2_ref, b2_ref, wn_ref, tq_ref,
                noise_ref, scal_ref, st_ref):
    xt = x_ref[...]
    HK = 512
    q = jnp.broadcast_to(b2_ref[...], (xt.shape[0], N_EMBD))
    for k in range(N_HID // HK):
        hk = lax.dot_general(xt, w1_ref[k * HK:(k + 1) * HK, :],
                             (((1,), (1,)), ((), ())),
                             preferred_element_type=jnp.float32)
        hk = hk + b1_ref[0, k * HK:(k + 1) * HK]
        hk = 0.5 * hk * (1.0 + lax.erf(hk * np.float32(1.0 / np.sqrt(2.0))))
        q = q + lax.dot_general(hk, w2_ref[:, k * HK:(k + 1) * HK],
                                (((1,), (1,)), ((), ())),
                                preferred_element_type=jnp.float32)
    s = lax.dot_general(tq_ref[...], q, (((1,), (1,)), ((), ())),
                        preferred_element_type=jnp.float32)
    g = lax.dot_general(wn_ref[...], xt, (((1,), (1,)), ((), ())),
                        preferred_element_type=jnp.float32)
    temp = scal_ref[0]
    inv_tau = scal_ref[1]
    bn = scal_ref[2]
    gate = jax.nn.sigmoid(g + bn)
    st_ref[...] = (s + (temp * gate) * noise_ref[...]) * inv_tau


def _scores_tc(xf, W1, b1, W2, b2, Wn, type_queries, noise_t, scal):
    grid = (N_TOK // TILE,)
    return pl.pallas_call(
        _score_body,
        grid=grid,
        in_specs=[
            pl.BlockSpec((TILE, N_EMBD), lambda i: (i, 0)),
            pl.BlockSpec((N_HID, N_EMBD), lambda i: (0, 0)),
            pl.BlockSpec((1, N_HID), lambda i: (0, 0)),
            pl.BlockSpec((N_EMBD, N_HID), lambda i: (0, 0)),
            pl.BlockSpec((1, N_EMBD), lambda i: (0, 0)),
            pl.BlockSpec((1, N_EMBD), lambda i: (0, 0)),
            pl.BlockSpec((N_EXP, N_EMBD), lambda i: (0, 0)),
            pl.BlockSpec((N_EXP, TILE), lambda i: (0, i)),
            pl.BlockSpec(memory_space=pltpu.SMEM),
        ],
        out_specs=pl.BlockSpec((N_EXP, TILE), lambda i: (0, i)),
        out_shape=jax.ShapeDtypeStruct((N_EXP, N_TOK), jnp.float32),
    )(xf, W1, b1.reshape(1, N_HID), W2, b2.reshape(1, N_EMBD), Wn,
      type_queries, noise_t, scal)


@functools.partial(
    pl.kernel,
    mesh=plsc.VectorSubcoreMesh(core_axis_name="c", subcore_axis_name="s",
                                num_cores=NC, num_subcores=NS),
    out_type=[
        jax.ShapeDtypeStruct((N_TOK * N_EXP,), jnp.float32),
        jax.ShapeDtypeStruct((N_TOK * 2,), jnp.int32),
    ],
    scratch_types=[
        pltpu.VMEM((N_EXP, CH), jnp.float32),
        pltpu.VMEM((CH * N_EXP,), jnp.float32),
        pltpu.VMEM((CH * 2,), jnp.int32),
    ],
    compiler_params=pltpu.CompilerParams(needs_layout_passes=False,
                                         use_tc_tiling_on_sc=True),
)
def _sc_router(st_hbm, out_hbm, idx_hbm, buf, outbuf, idxbuf):
    wid = lax.axis_index("s") * NC + lax.axis_index("c")
    lane = lax.iota(jnp.int32, LANES)
    zv = jnp.zeros((LANES,), jnp.float32)
    neg = jnp.full((LANES,), -jnp.inf, jnp.float32)
    zi = jnp.zeros((LANES,), jnp.int32)
    for c in range(NCHUNK):
        tok0 = wid * TPW + c * CH
        pltpu.sync_copy(st_hbm.at[:, pl.ds(tok0, CH)], buf)

        def group_body(g, _):
            g16 = g * LANES

            def zero_body(j, _):
                outbuf[pl.ds(g16 * N_EXP + j * LANES, LANES)] = zv
                return 0

            lax.fori_loop(0, (LANES * N_EXP) // LANES, zero_body, 0,
                          unroll=8)

            def expert_body(e, carry):
                m1, i1, m2, i2 = carry
                v = buf[e, pl.ds(g16, LANES)]
                ev = jnp.full((LANES,), e, jnp.int32)
                ga = v > m1
                gb = v > m2
                m2n = jnp.where(ga, m1, jnp.where(gb, v, m2))
                i2n = jnp.where(ga, i1, jnp.where(gb, ev, i2))
                m1n = jnp.where(ga, v, m1)
                i1n = jnp.where(ga, ev, i1)
                return m1n, i1n, m2n, i2n

            m1, i1, m2, i2 = lax.fori_loop(0, N_EXP, expert_body,
                                           (neg, zi, neg, zi), unroll=4)
            e2 = jnp.exp(m2 - m1)
            den = 1.0 + e2
            w1 = 1.0 / den
            w2 = e2 / den
            obase = (g16 + lane) * N_EXP
            plsc.store_scatter(outbuf, [obase + i1], w1)
            plsc.store_scatter(outbuf, [obase + i2], w2)
            ibase = (g16 + lane) * 2
            plsc.store_scatter(idxbuf, [ibase], i1)
            plsc.store_scatter(idxbuf, [ibase + 1], i2)
            return 0

        lax.fori_loop(0, GPC, group_body, 0)
        pltpu.sync_copy(outbuf, out_hbm.at[pl.ds(tok0 * N_EXP, CH * N_EXP)])
        pltpu.sync_copy(idxbuf, idx_hbm.at[pl.ds(tok0 * 2, CH * 2)])


def kernel(x, W1, b1, W2, b2, Wn, bn, type_queries, temperature):
    Bsz, Tlen, C = x.shape
    xf = x.reshape(N_TOK, C)
    noise_t = jnp.asarray(_noise_const())
    temp = jnp.clip(temperature * (0.95 ** (Tlen // 1000)), 0.1, 1.0)
    scal = jnp.stack([temp, 1.0 / (temp + 1e-6), bn[0]]).astype(jnp.float32)

    st = _scores_tc(xf, W1, b1, W2, b2, Wn, type_queries, noise_t, scal)
    out_flat, idx_flat = _sc_router(st)
    return (out_flat.reshape(Bsz, Tlen, N_EXP),
            idx_flat.reshape(Bsz, Tlen, 2))


# speedup vs baseline: 1.0426x; 1.0426x over previous
import functools

import jax
import jax.numpy as jnp
import numpy as np
from jax import lax
from jax.experimental import pallas as pl
from jax.experimental.pallas import tpu as pltpu
from jax.experimental.pallas import tpu_sc as plsc

N_EMBD = 768
N_HID = 4 * N_EMBD
N_EXP = 64
_B, _T = 4, 8192
N_TOK = _B * _T

TILE = 1024


@functools.lru_cache(maxsize=1)
def _noise_const() -> np.ndarray:
    with jax.ensure_compile_time_eval():
        n = jax.random.normal(jax.random.key(42), (N_TOK, N_EXP),
                              dtype=jnp.float32)
    return np.ascontiguousarray(np.asarray(n).T)

NC, NS, LANES = 2, 16, 16
NW = NC * NS
TPW = N_TOK // NW
CH = 256
NCHUNK = TPW // CH
GPC = CH // LANES


def _score_body(x_ref, w1_ref, b1_ref, w2_ref, b2_ref, wn_ref, tq_ref,
                noise_ref, scal_ref, st_ref):
    xt = x_ref[...]
    h = lax.dot_general(xt, w1_ref[...], (((1,), (1,)), ((), ())),
                        preferred_element_type=jnp.float32)
    h = h + b1_ref[...]
    h = 0.5 * h * (1.0 + lax.erf(h * np.float32(1.0 / np.sqrt(2.0))))
    q = lax.dot_general(h, w2_ref[...], (((1,), (1,)), ((), ())),
                        preferred_element_type=jnp.float32)
    q = q + b2_ref[...]
    s = lax.dot_general(tq_ref[...], q, (((1,), (1,)), ((), ())),
                        preferred_element_type=jnp.float32)
    g = lax.dot_general(wn_ref[...], xt, (((1,), (1,)), ((), ())),
                        preferred_element_type=jnp.float32)
    temp = scal_ref[0]
    inv_tau = scal_ref[1]
    bn = scal_ref[2]
    gate = jax.nn.sigmoid(g + bn)
    st_ref[...] = (s + (temp * gate) * noise_ref[...]) * inv_tau


def _scores_tc(xf, W1, b1, W2, b2, Wn, type_queries, noise_t, scal):
    grid = (N_TOK // TILE,)
    return pl.pallas_call(
        _score_body,
        grid=grid,
        in_specs=[
            pl.BlockSpec((TILE, N_EMBD), lambda i: (i, 0)),
            pl.BlockSpec((N_HID, N_EMBD), lambda i: (0, 0)),
            pl.BlockSpec((1, N_HID), lambda i: (0, 0)),
            pl.BlockSpec((N_EMBD, N_HID), lambda i: (0, 0)),
            pl.BlockSpec((1, N_EMBD), lambda i: (0, 0)),
            pl.BlockSpec((1, N_EMBD), lambda i: (0, 0)),
            pl.BlockSpec((N_EXP, N_EMBD), lambda i: (0, 0)),
            pl.BlockSpec((N_EXP, TILE), lambda i: (0, i)),
            pl.BlockSpec(memory_space=pltpu.SMEM),
        ],
        out_specs=pl.BlockSpec((N_EXP, TILE), lambda i: (0, i)),
        out_shape=jax.ShapeDtypeStruct((N_EXP, N_TOK), jnp.float32),
    )(xf, W1, b1.reshape(1, N_HID), W2, b2.reshape(1, N_EMBD), Wn,
      type_queries, noise_t, scal)


@functools.partial(
    pl.kernel,
    mesh=plsc.VectorSubcoreMesh(core_axis_name="c", subcore_axis_name="s",
                                num_cores=NC, num_subcores=NS),
    out_type=[
        jax.ShapeDtypeStruct((N_TOK * N_EXP,), jnp.float32),
        jax.ShapeDtypeStruct((N_TOK * 2,), jnp.int32),
    ],
    scratch_types=[
        pltpu.VMEM((N_EXP, CH), jnp.float32),
        pltpu.VMEM((N_EXP, CH), jnp.float32),
        pltpu.VMEM((CH * N_EXP,), jnp.float32),
        pltpu.VMEM((CH * N_EXP,), jnp.float32),
        pltpu.VMEM((CH * 2,), jnp.int32),
        pltpu.VMEM((CH * 2,), jnp.int32),
        pltpu.SemaphoreType.DMA,
        pltpu.SemaphoreType.DMA,
        pltpu.SemaphoreType.DMA,
        pltpu.SemaphoreType.DMA,
        pltpu.SemaphoreType.DMA,
        pltpu.SemaphoreType.DMA,
    ],
    compiler_params=pltpu.CompilerParams(needs_layout_passes=False,
                                         use_tc_tiling_on_sc=True),
)
def _sc_router(st_hbm, out_hbm, idx_hbm, ib0, ib1, ob0, ob1, xb0, xb1,
               is0, is1, os0, os1, xs0, xs1):
    wid = lax.axis_index("s") * NC + lax.axis_index("c")
    ibufs, obufs, xbufs = (ib0, ib1), (ob0, ob1), (xb0, xb1)
    isems, osems, xsems = (is0, is1), (os0, os1), (xs0, xs1)
    lane = lax.iota(jnp.int32, LANES)
    zv = jnp.zeros((LANES,), jnp.float32)
    neg = jnp.full((LANES,), -jnp.inf, jnp.float32)
    zi = jnp.zeros((LANES,), jnp.int32)

    def in_cp(c):
        tok0 = wid * TPW + c * CH
        return pltpu.make_async_copy(st_hbm.at[:, pl.ds(tok0, CH)],
                                     ibufs[c % 2], isems[c % 2])

    def out_cps(c):
        tok0 = wid * TPW + c * CH
        b = c % 2
        return (
            pltpu.make_async_copy(
                obufs[b], out_hbm.at[pl.ds(tok0 * N_EXP, CH * N_EXP)],
                osems[b]),
            pltpu.make_async_copy(
                xbufs[b], idx_hbm.at[pl.ds(tok0 * 2, CH * 2)], xsems[b]),
        )

    def compute_chunk(buf, outbuf, idxbuf):
        def group_body(g, _):
            g16 = g * LANES

            def zero_body(j, _):
                outbuf[pl.ds(g16 * N_EXP + j * LANES, LANES)] = zv
                return 0

            lax.fori_loop(0, (LANES * N_EXP) // LANES, zero_body, 0,
                          unroll=8)

            def expert_body(e, carry):
                m1, i1, m2, i2 = carry
                v = buf[e, pl.ds(g16, LANES)]
                ev = jnp.full((LANES,), e, jnp.int32)
                ga = v > m1
                gb = v > m2
                m2n = jnp.where(ga, m1, jnp.where(gb, v, m2))
                i2n = jnp.where(ga, i1, jnp.where(gb, ev, i2))
                m1n = jnp.where(ga, v, m1)
                i1n = jnp.where(ga, ev, i1)
                return m1n, i1n, m2n, i2n

            m1, i1, m2, i2 = lax.fori_loop(0, N_EXP, expert_body,
                                           (neg, zi, neg, zi), unroll=4)
            e2 = jnp.exp(m2 - m1)
            den = 1.0 + e2
            w1 = 1.0 / den
            w2 = e2 / den
            obase = (g16 + lane) * N_EXP
            plsc.store_scatter(outbuf, [obase + i1], w1)
            plsc.store_scatter(outbuf, [obase + i2], w2)
            ibase = (g16 + lane) * 2
            plsc.store_scatter(idxbuf, [ibase], i1)
            plsc.store_scatter(idxbuf, [ibase + 1], i2)
            return 0

        lax.fori_loop(0, GPC, group_body, 0)

    in_cp(0).start()
    in_cp(1).start()
    for c in range(NCHUNK):
        b = c % 2
        in_cp(c).wait()
        if c >= 2:
            o, xx = out_cps(c - 2)
            o.wait()
            xx.wait()
        compute_chunk(ibufs[b], obufs[b], xbufs[b])
        if c + 2 < NCHUNK:
            in_cp(c + 2).start()
        o, xx = out_cps(c)
        o.start()
        xx.start()
    for c in (NCHUNK - 2, NCHUNK - 1):
        o, xx = out_cps(c)
        o.wait()
        xx.wait()


def kernel(x, W1, b1, W2, b2, Wn, bn, type_queries, temperature):
    Bsz, Tlen, C = x.shape
    xf = x.reshape(N_TOK, C)
    noise_t = jnp.asarray(_noise_const())
    temp = jnp.clip(temperature * (0.95 ** (Tlen // 1000)), 0.1, 1.0)
    scal = jnp.stack([temp, 1.0 / (temp + 1e-6), bn[0]]).astype(jnp.float32)

    st = _scores_tc(xf, W1, b1, W2, b2, Wn, type_queries, noise_t, scal)
    out_flat, idx_flat = _sc_router(st)
    return (out_flat.reshape(Bsz, Tlen, N_EXP),
            idx_flat.reshape(Bsz, Tlen, 2))
